# baseline (speedup 1.0000x reference)
import jax
import jax.numpy as jnp
from jax.experimental import pallas as pl
from jax.experimental.pallas import tpu as pltpu

_D = 768
_E = 8
_BT = 1024


def _router_body(x_ref, w_ref, b_ref, wout_ref, iout_ref):
    g = jnp.dot(x_ref[...], w_ref[...], preferred_element_type=jnp.float32)
    gt = g.T + b_ref[...]
    idx = jax.lax.broadcasted_iota(jnp.int32, gt.shape, 0)
    big = jnp.int32(_E)
    m1 = jnp.max(gt, axis=0, keepdims=True)
    i1 = jnp.min(jnp.where(gt == m1, idx, big), axis=0, keepdims=True)
    g2 = jnp.where(idx == i1, -jnp.inf, gt)
    m2 = jnp.max(g2, axis=0, keepdims=True)
    i2 = jnp.min(jnp.where(g2 == m2, idx, big), axis=0, keepdims=True)
    e = jnp.exp(m2 - m1)
    w1 = 1.0 / (1.0 + e)
    wout_ref[...] = jnp.concatenate([w1, 1.0 - w1], axis=0).T
    iout_ref[...] = jnp.concatenate([i1, i2], axis=0).T


def kernel(x, W, b):
    original_shape = x.shape
    T = x.shape[0] * x.shape[1]
    x_flat = x.reshape(T, _D)
    Wt = W.T
    bcol = b.reshape(_E, 1)
    grid = (T // _BT,)
    weights_t, indices_t = pl.pallas_call(
        _router_body,
        grid=grid,
        in_specs=[
            pl.BlockSpec((_BT, _D), lambda i: (i, 0)),
            pl.BlockSpec((_D, _E), lambda i: (0, 0)),
            pl.BlockSpec((_E, 1), lambda i: (0, 0)),
        ],
        out_specs=[
            pl.BlockSpec((_BT, 2), lambda i: (i, 0)),
            pl.BlockSpec((_BT, 2), lambda i: (i, 0)),
        ],
        out_shape=[
            jax.ShapeDtypeStruct((T, 2), jnp.float32),
            jax.ShapeDtypeStruct((T, 2), jnp.int32),
        ],
        compiler_params=pltpu.CompilerParams(
            dimension_semantics=("arbitrary",),
        ),
    )(x_flat, Wt, bcol)
    return (
        weights_t.reshape(*original_shape[:-1], 2),
        indices_t.reshape(*original_shape[:-1], 2),
    )


# speedup vs baseline: 2.1100x; 2.1100x over previous
import jax
import jax.numpy as jnp
from jax.experimental import pallas as pl
from jax.experimental.pallas import tpu as pltpu

_D = 768
_E = 8
_BT = 2048


def _router_body(x_ref, w_ref, b_ref, wout_ref, iout_ref):
    g = jnp.dot(x_ref[...], w_ref[...], preferred_element_type=jnp.float32)
    gt = g.T + b_ref[...]
    idx = jax.lax.broadcasted_iota(jnp.int32, gt.shape, 0)
    big = jnp.int32(_E)
    m1 = jnp.max(gt, axis=0, keepdims=True)
    i1 = jnp.min(jnp.where(gt == m1, idx, big), axis=0, keepdims=True)
    g2 = jnp.where(idx == i1, -jnp.inf, gt)
    m2 = jnp.max(g2, axis=0, keepdims=True)
    i2 = jnp.min(jnp.where(g2 == m2, idx, big), axis=0, keepdims=True)
    e = jnp.exp(m2 - m1)
    w1 = 1.0 / (1.0 + e)
    wout_ref[...] = jnp.concatenate([w1, 1.0 - w1], axis=0)
    iout_ref[...] = jnp.concatenate([i1, i2], axis=0)


def kernel(x, W, b):
    original_shape = x.shape
    T = x.shape[0] * x.shape[1]
    x_flat = x.reshape(T, _D)
    Wt = W.T
    bcol = b.reshape(_E, 1)
    grid = (T // _BT,)
    weights_t, indices_t = pl.pallas_call(
        _router_body,
        grid=grid,
        in_specs=[
            pl.BlockSpec((_BT, _D), lambda i: (i, 0)),
            pl.BlockSpec((_D, _E), lambda i: (0, 0)),
            pl.BlockSpec((_E, 1), lambda i: (0, 0)),
        ],
        out_specs=[
            pl.BlockSpec((2, _BT), lambda i: (0, i)),
            pl.BlockSpec((2, _BT), lambda i: (0, i)),
        ],
        out_shape=[
            jax.ShapeDtypeStruct((2, T), jnp.float32),
            jax.ShapeDtypeStruct((2, T), jnp.int32),
        ],
        compiler_params=pltpu.CompilerParams(
            dimension_semantics=("arbitrary",),
        ),
    )(x_flat, Wt, bcol)
    return (
        weights_t.T.reshape(*original_shape[:-1], 2),
        indices_t.T.reshape(*original_shape[:-1], 2),
    )


# speedup vs baseline: 2.1769x; 1.0317x over previous
import jax
import jax.numpy as jnp
from jax.experimental import pallas as pl
from jax.experimental.pallas import tpu as pltpu

_D = 768
_E = 8
_BT = 4096


def _router_body(x_ref, w_ref, b_ref, wout_ref, iout_ref):
    g = jnp.dot(x_ref[...], w_ref[...], preferred_element_type=jnp.float32)
    gt = g.T + b_ref[...]
    idx = jax.lax.broadcasted_iota(jnp.int32, gt.shape, 0)
    big = jnp.int32(_E)
    m1 = jnp.max(gt, axis=0, keepdims=True)
    i1 = jnp.min(jnp.where(gt == m1, idx, big), axis=0, keepdims=True)
    g2 = jnp.where(idx == i1, -jnp.inf, gt)
    m2 = jnp.max(g2, axis=0, keepdims=True)
    i2 = jnp.min(jnp.where(g2 == m2, idx, big), axis=0, keepdims=True)
    e = jnp.exp(m2 - m1)
    w1 = 1.0 / (1.0 + e)
    wout_ref[...] = jnp.concatenate([w1, 1.0 - w1], axis=0)
    iout_ref[...] = jnp.concatenate([i1, i2], axis=0)


def kernel(x, W, b):
    original_shape = x.shape
    T = x.shape[0] * x.shape[1]
    x_flat = x.reshape(T, _D)
    Wt = W.T
    bcol = b.reshape(_E, 1)
    grid = (T // _BT,)
    weights_t, indices_t = pl.pallas_call(
        _router_body,
        grid=grid,
        in_specs=[
            pl.BlockSpec((_BT, _D), lambda i: (i, 0)),
            pl.BlockSpec((_D, _E), lambda i: (0, 0)),
            pl.BlockSpec((_E, 1), lambda i: (0, 0)),
        ],
        out_specs=[
            pl.BlockSpec((2, _BT), lambda i: (0, i)),
            pl.BlockSpec((2, _BT), lambda i: (0, i)),
        ],
        out_shape=[
            jax.ShapeDtypeStruct((2, T), jnp.float32),
            jax.ShapeDtypeStruct((2, T), jnp.int32),
        ],
        compiler_params=pltpu.CompilerParams(
            dimension_semantics=("arbitrary",),
        ),
    )(x_flat, Wt, bcol)
    return (
        weights_t.T.reshape(*original_shape[:-1], 2),
        indices_t.T.reshape(*original_shape[:-1], 2),
    )
